# baseline (speedup 1.0000x reference)
import functools

import jax
import jax.numpy as jnp
from jax.experimental import pallas as pl
from jax.experimental.pallas import tpu as pltpu

_CW = 1024


def _chamfer_body(pred_ref, gt_ref, loss_ref, *, nb, n, m):
    b = pl.program_id(0)

    p = pred_ref[0]
    g = gt_ref[0]
    px = p[:, 0:1]
    py = p[:, 1:2]
    pz = p[:, 2:3]

    rowacc = jnp.full((n, _CW), jnp.inf, dtype=jnp.bfloat16)
    colsum = jnp.float32(0.0)
    for j in range(m // _CW):
        lo, hi = j * _CW, (j + 1) * _CW
        d = (jnp.abs(px - g[0:1, lo:hi])
             + jnp.abs(py - g[1:2, lo:hi])
             + jnp.abs(pz - g[2:3, lo:hi]))
        rowacc = jnp.minimum(rowacc, d)
        colsum += jnp.sum(jnp.min(d, axis=0).astype(jnp.float32))

    rowsum = jnp.sum(jnp.min(rowacc, axis=1).astype(jnp.float32))

    @pl.when(b == 0)
    def _():
        loss_ref[0, 0] = 0.0

    loss_ref[0, 0] += rowsum / (n * nb) + colsum / (m * nb)


def kernel(pred, gt):
    nb, n, _ = pred.shape
    m = gt.shape[1]
    pred_bf = pred.astype(jnp.bfloat16)
    gt_t = jnp.transpose(gt.astype(jnp.bfloat16), (0, 2, 1))

    body = functools.partial(_chamfer_body, nb=nb, n=n, m=m)
    loss = pl.pallas_call(
        body,
        grid=(nb,),
        in_specs=[
            pl.BlockSpec((1, n, 3), lambda b: (b, 0, 0)),
            pl.BlockSpec((1, 3, m), lambda b: (b, 0, 0)),
        ],
        out_specs=pl.BlockSpec(
            (1, 1), lambda b: (0, 0), memory_space=pltpu.SMEM
        ),
        out_shape=jax.ShapeDtypeStruct((1, 1), jnp.float32),
    )(pred_bf, gt_t)
    return loss[0, 0]


# speedup vs baseline: 1.0846x; 1.0846x over previous
import functools

import jax
import jax.numpy as jnp
from jax.experimental import pallas as pl
from jax.experimental.pallas import tpu as pltpu

_CW = 1024


def _chamfer_body(pred_ref, gt_ref, loss_ref, *, nb, n, m):
    b = pl.program_id(0)

    p = pred_ref[0].astype(jnp.bfloat16)
    g = gt_ref[0].astype(jnp.bfloat16)
    px = p[:, 0:1]
    py = p[:, 1:2]
    pz = p[:, 2:3]

    rowacc = jnp.full((n, _CW), jnp.inf, dtype=jnp.bfloat16)
    colsum = jnp.float32(0.0)
    for j in range(m // _CW):
        lo, hi = j * _CW, (j + 1) * _CW
        d = (jnp.abs(px - g[0:1, lo:hi])
             + jnp.abs(py - g[1:2, lo:hi])
             + jnp.abs(pz - g[2:3, lo:hi]))
        rowacc = jnp.minimum(rowacc, d)
        colsum += jnp.sum(jnp.min(d, axis=0).astype(jnp.float32))

    rowsum = jnp.sum(jnp.min(rowacc, axis=1).astype(jnp.float32))

    @pl.when(b == 0)
    def _():
        loss_ref[0, 0] = 0.0

    loss_ref[0, 0] += rowsum / (n * nb) + colsum / (m * nb)


def kernel(pred, gt):
    nb, n, _ = pred.shape
    m = gt.shape[1]
    gt_t = jnp.transpose(gt, (0, 2, 1))

    body = functools.partial(_chamfer_body, nb=nb, n=n, m=m)
    loss = pl.pallas_call(
        body,
        grid=(nb,),
        in_specs=[
            pl.BlockSpec((1, n, 3), lambda b: (b, 0, 0)),
            pl.BlockSpec((1, 3, m), lambda b: (b, 0, 0)),
        ],
        out_specs=pl.BlockSpec(
            (1, 1), lambda b: (0, 0), memory_space=pltpu.SMEM
        ),
        out_shape=jax.ShapeDtypeStruct((1, 1), jnp.float32),
    )(pred, gt_t)
    return loss[0, 0]
